# baseline (speedup 1.0000x reference)
import jax
import jax.numpy as jnp
from jax import lax
from jax.experimental import pallas as pl
from jax.experimental.pallas import tpu as pltpu
from jax.experimental.pallas import tpu_sc as plsc

CUTOFF = 5.0
KB = 2560
LANES = 16

_PI = 3.141592653589793
C0 = _PI
C1 = -(_PI**3) / 6.0
C2 = (_PI**5) / 120.0
C3 = -(_PI**7) / 5040.0
C4 = (_PI**9) / 362880.0


def _splat(x, dtype=jnp.float32):
    return lax.broadcast_in_dim(jnp.asarray(x, dtype), (LANES,), ())


def _num_workers():
    try:
        info = plsc.get_sparse_core_info()
        return info.num_cores, info.num_subcores
    except Exception:
        return 2, 16


def kernel(vec, distances, filter_indices):
    e_in = vec.shape[0]
    e_f = filter_indices.shape[0]
    nc, ns = _num_workers()
    nw = nc * ns
    assert e_f % KB == 0, e_f
    nblk = e_f // KB
    tmax = pl.cdiv(nblk, nw)
    ngrp = KB // LANES

    def body(vec_hbm, dist_hbm, fidx_hbm, vecf_hbm, distf_hbm, sw_hbm,
             mask_hbm, idx_b, cidxd_b, cidx0_b, cidx1_b, cidx2_b, dist_b,
             vc0_b, vc1_b, vc2_b, vstage_b, sw_b, mask_b, sem):
        wid = lax.axis_index("s") * nc + lax.axis_index("c")
        iota = lax.iota(jnp.int32, LANES)
        e_in_v = _splat(e_in, jnp.int32)
        zero_i = _splat(0, jnp.int32)
        one_i = _splat(1, jnp.int32)
        neg1 = _splat(-1, jnp.int32)
        fill_v = _splat(CUTOFF)
        zero_f = _splat(0.0)
        half_v = _splat(0.5)
        inv_cut = _splat(1.0 / CUTOFF)
        c0 = _splat(C0)
        c1 = _splat(C1)
        c2 = _splat(C2)
        c3 = _splat(C3)
        c4 = _splat(C4)
        bitvals = [_splat(1 << (8 * k), jnp.int32) for k in range(4)]

        def block_body(t, carry):
            b = wid + nw * t

            @pl.when(b < nblk)
            def _():
                base = b * KB
                pltpu.sync_copy(fidx_hbm.at[pl.ds(base, KB)], idx_b)

                def clamp_body(j, c):
                    s16 = pl.ds(j * LANES, LANES)
                    iv = idx_b[s16]
                    inb = iv < e_in_v
                    i3 = iv * 3
                    cidxd_b[s16] = jnp.where(inb, iv, neg1)
                    cidx0_b[s16] = jnp.where(inb, i3, neg1)
                    cidx1_b[s16] = jnp.where(inb, i3 + one_i, neg1)
                    cidx2_b[s16] = jnp.where(inb, i3 + one_i + one_i, neg1)
                    return c

                lax.fori_loop(0, ngrp, clamp_body, 0)

                cp = [
                    pltpu.async_copy(
                        dist_hbm.at[plsc.Indices(cidxd_b, ignored_value=-1)],
                        dist_b, sem),
                    pltpu.async_copy(
                        vec_hbm.at[plsc.Indices(cidx0_b, ignored_value=-1)],
                        vc0_b, sem),
                    pltpu.async_copy(
                        vec_hbm.at[plsc.Indices(cidx1_b, ignored_value=-1)],
                        vc1_b, sem),
                    pltpu.async_copy(
                        vec_hbm.at[plsc.Indices(cidx2_b, ignored_value=-1)],
                        vc2_b, sem),
                ]
                for c in cp:
                    c.wait()

                def post_body(j, c):
                    s16 = pl.ds(j * LANES, LANES)
                    iv = idx_b[s16]
                    oob = iv >= e_in_v
                    d = jnp.where(oob, fill_v, dist_b[s16])
                    dist_b[s16] = d
                    edge = d < fill_v
                    u = d * inv_cut - half_v
                    u2 = u * u
                    p = (((c4 * u2 + c3) * u2 + c2) * u2 + c1) * u2 + c0
                    sw = half_v - (half_v * u) * p
                    sw_b[s16] = jnp.where(edge, sw, zero_f)
                    f0 = (iota + lax.broadcast_in_dim(
                        j * LANES, (LANES,), ())) * 3
                    v0 = jnp.where(oob, fill_v, vc0_b[s16])
                    v1 = jnp.where(oob, fill_v, vc1_b[s16])
                    v2 = jnp.where(oob, fill_v, vc2_b[s16])
                    plsc.store_scatter(vstage_b, [f0], v0)
                    plsc.store_scatter(vstage_b, [f0 + one_i], v1)
                    plsc.store_scatter(vstage_b, [f0 + one_i + one_i], v2)
                    return c

                lax.fori_loop(0, ngrp, post_body, 0)

                def mask_body(q, c):
                    base64 = q * (4 * LANES)
                    w = zero_i
                    for k in range(4):
                        gidx = iota * 4 + lax.broadcast_in_dim(
                            base64 + k, (LANES,), ())
                        dk = plsc.load_gather(dist_b, [gidx])
                        mk = dk < fill_v
                        w = w | jnp.where(mk, bitvals[k], zero_i)
                    mask_b[pl.ds(base64, 4 * LANES)] = plsc.bitcast(
                        w, jnp.int8)
                    return c

                lax.fori_loop(0, ngrp // 4, mask_body, 0)

                pltpu.sync_copy(dist_b, distf_hbm.at[pl.ds(base, KB)])
                pltpu.sync_copy(sw_b, sw_hbm.at[pl.ds(base, KB)])
                pltpu.sync_copy(vstage_b,
                                vecf_hbm.at[pl.ds(base * 3, KB * 3)])
                pltpu.sync_copy(mask_b, mask_hbm.at[pl.ds(base, KB)])

            return carry

        lax.fori_loop(0, tmax, block_body, 0)

    mesh = plsc.VectorSubcoreMesh(core_axis_name="c", subcore_axis_name="s")
    run = pl.kernel(
        body,
        out_type=(
            jax.ShapeDtypeStruct((e_f * 3,), jnp.float32),
            jax.ShapeDtypeStruct((e_f,), jnp.float32),
            jax.ShapeDtypeStruct((e_f,), jnp.float32),
            jax.ShapeDtypeStruct((e_f,), jnp.int8),
        ),
        mesh=mesh,
        compiler_params=pltpu.CompilerParams(
            needs_layout_passes=False, use_tc_tiling_on_sc=False),
        scratch_types=[
            pltpu.VMEM((KB,), jnp.int32),
            pltpu.VMEM((KB,), jnp.int32),
            pltpu.VMEM((KB,), jnp.int32),
            pltpu.VMEM((KB,), jnp.int32),
            pltpu.VMEM((KB,), jnp.int32),
            pltpu.VMEM((KB,), jnp.float32),
            pltpu.VMEM((KB,), jnp.float32),
            pltpu.VMEM((KB,), jnp.float32),
            pltpu.VMEM((KB,), jnp.float32),
            pltpu.VMEM((KB * 3,), jnp.float32),
            pltpu.VMEM((KB,), jnp.float32),
            pltpu.VMEM((KB,), jnp.int8),
            pltpu.SemaphoreType.DMA,
        ],
    )
    vec_f, dist_f, switch, mask8 = run(vec.reshape(-1), distances,
                                       filter_indices)
    return (vec_f.reshape(e_f, 3), dist_f, switch,
            mask8.astype(jnp.bool_))


# speedup vs baseline: 1.0004x; 1.0004x over previous
import jax
import jax.numpy as jnp
from jax import lax
from jax.experimental import pallas as pl
from jax.experimental.pallas import tpu as pltpu
from jax.experimental.pallas import tpu_sc as plsc

CUTOFF = 5.0
KB = 2560
LANES = 16

_PI = 3.141592653589793
C0 = _PI
C1 = -(_PI**3) / 6.0
C2 = (_PI**5) / 120.0
C3 = -(_PI**7) / 5040.0
C4 = (_PI**9) / 362880.0


def _splat(x, dtype=jnp.float32):
    return lax.broadcast_in_dim(jnp.asarray(x, dtype), (LANES,), ())


def _num_workers():
    try:
        info = plsc.get_sparse_core_info()
        return info.num_cores, info.num_subcores
    except Exception:
        return 2, 16


def kernel(vec, distances, filter_indices):
    e_in = vec.shape[0]
    e_f = filter_indices.shape[0]
    nc, ns = _num_workers()
    nw = nc * ns
    assert e_f % KB == 0, e_f
    nblk = e_f // KB
    tmax = pl.cdiv(nblk, nw)
    ngrp = KB // LANES

    def body(vec_hbm, dist_hbm, fidx_hbm, vecf_hbm, distf_hbm, sw_hbm,
             mask_hbm, idx_b, cidxd_b, cidx0_b, cidx1_b, cidx2_b, dist_b,
             vc0_b, vc1_b, vc2_b, vstage_b, sw_b, mask_b, sem):
        wid = lax.axis_index("s") * nc + lax.axis_index("c")
        iota = lax.iota(jnp.int32, LANES)
        e_in_v = _splat(e_in, jnp.int32)
        zero_i = _splat(0, jnp.int32)
        one_i = _splat(1, jnp.int32)
        neg1 = _splat(-1, jnp.int32)
        fill_v = _splat(CUTOFF)
        zero_f = _splat(0.0)
        half_v = _splat(0.5)
        inv_cut = _splat(1.0 / CUTOFF)
        c0 = _splat(C0)
        c1 = _splat(C1)
        c2 = _splat(C2)
        c3 = _splat(C3)
        c4 = _splat(C4)
        bitvals = [_splat(1 << (8 * k), jnp.int32) for k in range(4)]

        def block_body(t, carry):
            b = wid + nw * t

            @pl.when(b < nblk)
            def _():
                base = b * KB
                pltpu.sync_copy(fidx_hbm.at[pl.ds(base, KB)], idx_b)

                def clamp_body(j, c):
                    s16 = pl.ds(j * LANES, LANES)
                    iv = idx_b[s16]
                    inb = iv < e_in_v
                    i3 = iv * 3
                    cidxd_b[s16] = jnp.where(inb, iv, neg1)
                    cidx0_b[s16] = jnp.where(inb, i3, neg1)
                    cidx1_b[s16] = jnp.where(inb, i3 + one_i, neg1)
                    cidx2_b[s16] = jnp.where(inb, i3 + one_i + one_i, neg1)
                    return c

                lax.fori_loop(0, ngrp, clamp_body, 0)

                cp = [
                    pltpu.async_copy(
                        dist_hbm.at[plsc.Indices(cidxd_b, ignored_value=-1)],
                        dist_b, sem),
                    pltpu.async_copy(
                        vec_hbm.at[plsc.Indices(cidx0_b, ignored_value=-1)],
                        vc0_b, sem),
                    pltpu.async_copy(
                        vec_hbm.at[plsc.Indices(cidx1_b, ignored_value=-1)],
                        vc1_b, sem),
                    pltpu.async_copy(
                        vec_hbm.at[plsc.Indices(cidx2_b, ignored_value=-1)],
                        vc2_b, sem),
                ]
                for c in cp:
                    c.wait()

                def post_body(j, c):
                    s16 = pl.ds(j * LANES, LANES)
                    iv = idx_b[s16]
                    oob = iv >= e_in_v
                    d = jnp.where(oob, fill_v, dist_b[s16])
                    dist_b[s16] = d
                    edge = d < fill_v
                    u = d * inv_cut - half_v
                    u2 = u * u
                    p = (((c4 * u2 + c3) * u2 + c2) * u2 + c1) * u2 + c0
                    sw = half_v - (half_v * u) * p
                    sw_b[s16] = jnp.where(edge, sw, zero_f)
                    f0 = (iota + lax.broadcast_in_dim(
                        j * LANES, (LANES,), ())) * 3
                    v0 = jnp.where(oob, fill_v, vc0_b[s16])
                    v1 = jnp.where(oob, fill_v, vc1_b[s16])
                    v2 = jnp.where(oob, fill_v, vc2_b[s16])
                    plsc.store_scatter(vstage_b, [f0], v0)
                    plsc.store_scatter(vstage_b, [f0 + one_i], v1)
                    plsc.store_scatter(vstage_b, [f0 + one_i + one_i], v2)
                    return c

                lax.fori_loop(0, ngrp, post_body, 0)

                def mask_body(q, c):
                    base64 = q * (4 * LANES)
                    w = zero_i
                    for k in range(4):
                        gidx = iota * 4 + lax.broadcast_in_dim(
                            base64 + k, (LANES,), ())
                        dk = plsc.load_gather(dist_b, [gidx])
                        mk = dk < fill_v
                        w = w | jnp.where(mk, bitvals[k], zero_i)
                    mask_b[pl.ds(q * LANES, LANES)] = w
                    return c

                lax.fori_loop(0, ngrp // 4, mask_body, 0)

                pltpu.sync_copy(dist_b, distf_hbm.at[pl.ds(base, KB)])
                pltpu.sync_copy(sw_b, sw_hbm.at[pl.ds(base, KB)])
                pltpu.sync_copy(vstage_b,
                                vecf_hbm.at[pl.ds(base * 3, KB * 3)])
                pltpu.sync_copy(mask_b,
                                mask_hbm.at[pl.ds(b * (KB // 4), KB // 4)])

            return carry

        lax.fori_loop(0, tmax, block_body, 0)

    mesh = plsc.VectorSubcoreMesh(core_axis_name="c", subcore_axis_name="s")
    run = pl.kernel(
        body,
        out_type=(
            jax.ShapeDtypeStruct((e_f * 3,), jnp.float32),
            jax.ShapeDtypeStruct((e_f,), jnp.float32),
            jax.ShapeDtypeStruct((e_f,), jnp.float32),
            jax.ShapeDtypeStruct((e_f // 4,), jnp.int32),
        ),
        mesh=mesh,
        compiler_params=pltpu.CompilerParams(needs_layout_passes=False),
        scratch_types=[
            pltpu.VMEM((KB,), jnp.int32),
            pltpu.VMEM((KB,), jnp.int32),
            pltpu.VMEM((KB,), jnp.int32),
            pltpu.VMEM((KB,), jnp.int32),
            pltpu.VMEM((KB,), jnp.int32),
            pltpu.VMEM((KB,), jnp.float32),
            pltpu.VMEM((KB,), jnp.float32),
            pltpu.VMEM((KB,), jnp.float32),
            pltpu.VMEM((KB,), jnp.float32),
            pltpu.VMEM((KB * 3,), jnp.float32),
            pltpu.VMEM((KB,), jnp.float32),
            pltpu.VMEM((KB // 4,), jnp.int32),
            pltpu.SemaphoreType.DMA,
        ],
    )
    vec_f, dist_f, switch, maskw = run(vec.reshape(-1), distances,
                                       filter_indices)
    mask8 = lax.bitcast_convert_type(maskw, jnp.uint8).reshape(e_f)
    return (vec_f.reshape(e_f, 3), dist_f, switch,
            mask8.astype(jnp.bool_))


# speedup vs baseline: 3.4139x; 3.4126x over previous
import jax
import jax.numpy as jnp
from jax import lax
from jax.experimental import pallas as pl
from jax.experimental.pallas import tpu as pltpu
from jax.experimental.pallas import tpu_sc as plsc

CUTOFF = 5.0
KB = 2560
LANES = 16

_PI = 3.141592653589793
C0 = _PI
C1 = -(_PI**3) / 6.0
C2 = (_PI**5) / 120.0
C3 = -(_PI**7) / 5040.0
C4 = (_PI**9) / 362880.0


def _splat(x, dtype=jnp.float32):
    return lax.broadcast_in_dim(jnp.asarray(x, dtype), (LANES,), ())


def _num_workers():
    try:
        info = plsc.get_sparse_core_info()
        return info.num_cores, info.num_subcores
    except Exception:
        return 2, 16


def kernel(vec, distances, filter_indices):
    e_in = vec.shape[0]
    e_f = filter_indices.shape[0]
    nc, ns = _num_workers()
    nw = nc * ns
    assert e_f % KB == 0, e_f
    nblk = e_f // KB
    tmax = pl.cdiv(nblk, nw)
    ngrp = KB // LANES

    def body(vec_hbm, dist_hbm, fidx_hbm, vecf_hbm, distf_hbm, sw_hbm,
             mask_hbm, idx_b, cidxd_b, cidx0_b, cidx1_b, cidx2_b, dist_b,
             vc0_b, vc1_b, vc2_b, sw_b, mask_b, sem):
        wid = lax.axis_index("s") * nc + lax.axis_index("c")
        iota = lax.iota(jnp.int32, LANES)
        e_in_v = _splat(e_in, jnp.int32)
        zero_i = _splat(0, jnp.int32)
        one_i = _splat(1, jnp.int32)
        neg1 = _splat(-1, jnp.int32)
        fill_v = _splat(CUTOFF)
        zero_f = _splat(0.0)
        half_v = _splat(0.5)
        inv_cut = _splat(1.0 / CUTOFF)
        c0 = _splat(C0)
        c1 = _splat(C1)
        c2 = _splat(C2)
        c3 = _splat(C3)
        c4 = _splat(C4)
        bitvals = [_splat(1 << (8 * k), jnp.int32) for k in range(4)]

        def block_body(t, carry):
            b = wid + nw * t

            @pl.when(b < nblk)
            def _():
                base = b * KB
                pltpu.sync_copy(fidx_hbm.at[pl.ds(base, KB)], idx_b)

                def clamp_body(j, c):
                    s16 = pl.ds(j * LANES, LANES)
                    iv = idx_b[s16]
                    inb = iv < e_in_v
                    civ = jnp.where(inb, iv, neg1)
                    cidxd_b[s16] = civ
                    cidx0_b[s16] = civ
                    cidx1_b[s16] = jnp.where(inb, iv + e_in_v, neg1)
                    cidx2_b[s16] = jnp.where(inb, iv + e_in_v + e_in_v,
                                             neg1)
                    return c

                lax.fori_loop(0, ngrp, clamp_body, 0)

                cp = [
                    pltpu.async_copy(
                        dist_hbm.at[plsc.Indices(cidxd_b, ignored_value=-1)],
                        dist_b, sem),
                    pltpu.async_copy(
                        vec_hbm.at[plsc.Indices(cidx0_b, ignored_value=-1)],
                        vc0_b, sem),
                    pltpu.async_copy(
                        vec_hbm.at[plsc.Indices(cidx1_b, ignored_value=-1)],
                        vc1_b, sem),
                    pltpu.async_copy(
                        vec_hbm.at[plsc.Indices(cidx2_b, ignored_value=-1)],
                        vc2_b, sem),
                ]
                for c in cp:
                    c.wait()

                def post_body(j, c):
                    s16 = pl.ds(j * LANES, LANES)
                    iv = idx_b[s16]
                    oob = iv >= e_in_v
                    d = jnp.where(oob, fill_v, dist_b[s16])
                    dist_b[s16] = d
                    edge = d < fill_v
                    u = d * inv_cut - half_v
                    u2 = u * u
                    p = (((c4 * u2 + c3) * u2 + c2) * u2 + c1) * u2 + c0
                    sw = half_v - (half_v * u) * p
                    sw_b[s16] = jnp.where(edge, sw, zero_f)
                    vc0_b[s16] = jnp.where(oob, fill_v, vc0_b[s16])
                    vc1_b[s16] = jnp.where(oob, fill_v, vc1_b[s16])
                    vc2_b[s16] = jnp.where(oob, fill_v, vc2_b[s16])
                    return c

                lax.fori_loop(0, ngrp, post_body, 0)

                def mask_body(q, c):
                    base64 = q * (4 * LANES)
                    w = zero_i
                    for k in range(4):
                        gidx = iota * 4 + lax.broadcast_in_dim(
                            base64 + k, (LANES,), ())
                        dk = plsc.load_gather(dist_b, [gidx])
                        mk = dk < fill_v
                        w = w | jnp.where(mk, bitvals[k], zero_i)
                    mask_b[pl.ds(q * LANES, LANES)] = w
                    return c

                lax.fori_loop(0, ngrp // 4, mask_body, 0)

                pltpu.sync_copy(dist_b, distf_hbm.at[pl.ds(base, KB)])
                pltpu.sync_copy(sw_b, sw_hbm.at[pl.ds(base, KB)])
                pltpu.sync_copy(vc0_b, vecf_hbm.at[pl.ds(base, KB)])
                pltpu.sync_copy(vc1_b,
                                vecf_hbm.at[pl.ds(e_f + base, KB)])
                pltpu.sync_copy(vc2_b,
                                vecf_hbm.at[pl.ds(2 * e_f + base, KB)])
                pltpu.sync_copy(mask_b,
                                mask_hbm.at[pl.ds(b * (KB // 4), KB // 4)])

            return carry

        lax.fori_loop(0, tmax, block_body, 0)

    mesh = plsc.VectorSubcoreMesh(core_axis_name="c", subcore_axis_name="s")
    run = pl.kernel(
        body,
        out_type=(
            jax.ShapeDtypeStruct((e_f * 3,), jnp.float32),
            jax.ShapeDtypeStruct((e_f,), jnp.float32),
            jax.ShapeDtypeStruct((e_f,), jnp.float32),
            jax.ShapeDtypeStruct((e_f // 4,), jnp.int32),
        ),
        mesh=mesh,
        compiler_params=pltpu.CompilerParams(needs_layout_passes=False),
        scratch_types=[
            pltpu.VMEM((KB,), jnp.int32),
            pltpu.VMEM((KB,), jnp.int32),
            pltpu.VMEM((KB,), jnp.int32),
            pltpu.VMEM((KB,), jnp.int32),
            pltpu.VMEM((KB,), jnp.int32),
            pltpu.VMEM((KB,), jnp.float32),
            pltpu.VMEM((KB,), jnp.float32),
            pltpu.VMEM((KB,), jnp.float32),
            pltpu.VMEM((KB,), jnp.float32),
            pltpu.VMEM((KB,), jnp.float32),
            pltpu.VMEM((KB // 4,), jnp.int32),
            pltpu.SemaphoreType.DMA,
        ],
    )
    vec_f, dist_f, switch, maskw = run(vec.T.reshape(-1), distances,
                                       filter_indices)
    mask8 = lax.bitcast_convert_type(maskw, jnp.uint8).reshape(e_f)
    return (vec_f.reshape(3, e_f).T, dist_f, switch,
            mask8.astype(jnp.bool_))


# speedup vs baseline: 14.7989x; 4.3349x over previous
import jax
import jax.numpy as jnp
from jax import lax
from jax.experimental import pallas as pl
from jax.experimental.pallas import tpu as pltpu
from jax.experimental.pallas import tpu_sc as plsc

CUTOFF = 5.0
KB = 2560
LANES = 16
TC_BLK = 128000


def _splat(x, dtype=jnp.float32):
    return lax.broadcast_in_dim(jnp.asarray(x, dtype), (LANES,), ())


def _num_workers():
    try:
        info = plsc.get_sparse_core_info()
        return info.num_cores, info.num_subcores
    except Exception:
        return 2, 16


def _make_splitter(e_in):
    grid = pl.cdiv(e_in, TC_BLK)

    def split_body(vt_ref, p0_ref, p1_ref, p2_ref):
        x = vt_ref[...]
        p0_ref[...] = x[0]
        p1_ref[...] = x[1]
        p2_ref[...] = x[2]

    return pl.pallas_call(
        split_body,
        grid=(grid,),
        in_specs=[pl.BlockSpec((3, TC_BLK), lambda i: (0, i))],
        out_specs=[
            pl.BlockSpec((TC_BLK,), lambda i: (i,)),
            pl.BlockSpec((TC_BLK,), lambda i: (i,)),
            pl.BlockSpec((TC_BLK,), lambda i: (i,)),
        ],
        out_shape=[jax.ShapeDtypeStruct((e_in,), jnp.float32)] * 3,
    )


def _make_merger(e_f):
    grid = pl.cdiv(e_f, TC_BLK)

    def merge_body(v0_ref, v1_ref, v2_ref, d_ref, vt_ref, sw_ref, m_ref):
        vt_ref[...] = jnp.stack(
            [v0_ref[...], v1_ref[...], v2_ref[...]], axis=0)
        d = d_ref[...]
        edge = d < CUTOFF
        sw = 0.5 * jnp.cos(jnp.pi * (d * (1.0 / CUTOFF))) + 0.5
        sw_ref[...] = jnp.where(edge, sw, 0.0)
        m_ref[...] = edge

    return pl.pallas_call(
        merge_body,
        grid=(grid,),
        in_specs=[pl.BlockSpec((TC_BLK,), lambda i: (i,))] * 4,
        out_specs=[
            pl.BlockSpec((3, TC_BLK), lambda i: (0, i)),
            pl.BlockSpec((TC_BLK,), lambda i: (i,)),
            pl.BlockSpec((TC_BLK,), lambda i: (i,)),
        ],
        out_shape=[
            jax.ShapeDtypeStruct((3, e_f), jnp.float32),
            jax.ShapeDtypeStruct((e_f,), jnp.float32),
            jax.ShapeDtypeStruct((e_f,), jnp.bool_),
        ],
    )


def kernel(vec, distances, filter_indices):
    e_in = vec.shape[0]
    e_f = filter_indices.shape[0]
    nc, ns = _num_workers()
    nw = nc * ns
    assert e_f % KB == 0, e_f
    nblk = e_f // KB
    tmax = pl.cdiv(nblk, nw)
    ngrp = KB // LANES

    def body(p0_hbm, p1_hbm, p2_hbm, dist_hbm, fidx_hbm, vf0_hbm, vf1_hbm,
             vf2_hbm, distf_hbm, idx_b, cidx_b, dist_b, vc0_b, vc1_b,
             vc2_b, sem):
        wid = lax.axis_index("s") * nc + lax.axis_index("c")
        e_in_v = _splat(e_in, jnp.int32)
        neg1 = _splat(-1, jnp.int32)
        fill_v = _splat(CUTOFF)

        def block_body(t, carry):
            b = wid + nw * t

            @pl.when(b < nblk)
            def _():
                base = b * KB
                pltpu.sync_copy(fidx_hbm.at[pl.ds(base, KB)], idx_b)

                def clamp_body(j, c):
                    s16 = pl.ds(j * LANES, LANES)
                    iv = idx_b[s16]
                    cidx_b[s16] = jnp.where(iv < e_in_v, iv, neg1)
                    return c

                lax.fori_loop(0, ngrp, clamp_body, 0)

                idx = plsc.Indices(cidx_b, ignored_value=-1)
                cp = [
                    pltpu.async_copy(dist_hbm.at[idx], dist_b, sem),
                    pltpu.async_copy(p0_hbm.at[idx], vc0_b, sem),
                    pltpu.async_copy(p1_hbm.at[idx], vc1_b, sem),
                    pltpu.async_copy(p2_hbm.at[idx], vc2_b, sem),
                ]
                for c in cp:
                    c.wait()

                def post_body(j, c):
                    s16 = pl.ds(j * LANES, LANES)
                    oob = idx_b[s16] >= e_in_v
                    dist_b[s16] = jnp.where(oob, fill_v, dist_b[s16])
                    vc0_b[s16] = jnp.where(oob, fill_v, vc0_b[s16])
                    vc1_b[s16] = jnp.where(oob, fill_v, vc1_b[s16])
                    vc2_b[s16] = jnp.where(oob, fill_v, vc2_b[s16])
                    return c

                lax.fori_loop(0, ngrp, post_body, 0)

                pltpu.sync_copy(dist_b, distf_hbm.at[pl.ds(base, KB)])
                pltpu.sync_copy(vc0_b, vf0_hbm.at[pl.ds(base, KB)])
                pltpu.sync_copy(vc1_b, vf1_hbm.at[pl.ds(base, KB)])
                pltpu.sync_copy(vc2_b, vf2_hbm.at[pl.ds(base, KB)])

            return carry

        lax.fori_loop(0, tmax, block_body, 0)

    mesh = plsc.VectorSubcoreMesh(core_axis_name="c", subcore_axis_name="s")
    run = pl.kernel(
        body,
        out_type=(
            jax.ShapeDtypeStruct((e_f,), jnp.float32),
            jax.ShapeDtypeStruct((e_f,), jnp.float32),
            jax.ShapeDtypeStruct((e_f,), jnp.float32),
            jax.ShapeDtypeStruct((e_f,), jnp.float32),
        ),
        mesh=mesh,
        compiler_params=pltpu.CompilerParams(needs_layout_passes=False),
        scratch_types=[
            pltpu.VMEM((KB,), jnp.int32),
            pltpu.VMEM((KB,), jnp.int32),
            pltpu.VMEM((KB,), jnp.float32),
            pltpu.VMEM((KB,), jnp.float32),
            pltpu.VMEM((KB,), jnp.float32),
            pltpu.VMEM((KB,), jnp.float32),
            pltpu.SemaphoreType.DMA,
        ],
    )
    p0, p1, p2 = _make_splitter(e_in)(vec.T)
    vf0, vf1, vf2, dist_f = run(p0, p1, p2, distances, filter_indices)
    vecf_t, switch, mask = _make_merger(e_f)(vf0, vf1, vf2, dist_f)
    return vecf_t.T, dist_f, switch, mask


# speedup vs baseline: 15.6167x; 1.0553x over previous
import jax
import jax.numpy as jnp
from jax import lax
from jax.experimental import pallas as pl
from jax.experimental.pallas import tpu as pltpu
from jax.experimental.pallas import tpu_sc as plsc

CUTOFF = 5.0
KB = 6400
LANES = 16
TC_BLK = 128000


def _splat(x, dtype=jnp.float32):
    return lax.broadcast_in_dim(jnp.asarray(x, dtype), (LANES,), ())


def _num_workers():
    try:
        info = plsc.get_sparse_core_info()
        return info.num_cores, info.num_subcores
    except Exception:
        return 2, 16


def _make_splitter(e_in):
    grid = pl.cdiv(e_in, TC_BLK)

    def split_body(vt_ref, p0_ref, p1_ref, p2_ref):
        x = vt_ref[...]
        p0_ref[...] = x[0]
        p1_ref[...] = x[1]
        p2_ref[...] = x[2]

    return pl.pallas_call(
        split_body,
        grid=(grid,),
        in_specs=[pl.BlockSpec((3, TC_BLK), lambda i: (0, i))],
        out_specs=[
            pl.BlockSpec((TC_BLK,), lambda i: (i,)),
            pl.BlockSpec((TC_BLK,), lambda i: (i,)),
            pl.BlockSpec((TC_BLK,), lambda i: (i,)),
        ],
        out_shape=[jax.ShapeDtypeStruct((e_in,), jnp.float32)] * 3,
    )


def _make_merger(e_f):
    grid = pl.cdiv(e_f, TC_BLK)

    def merge_body(v0_ref, v1_ref, v2_ref, d_ref, vt_ref, sw_ref, m_ref):
        vt_ref[...] = jnp.stack(
            [v0_ref[...], v1_ref[...], v2_ref[...]], axis=0)
        d = d_ref[...]
        edge = d < CUTOFF
        sw = 0.5 * jnp.cos(jnp.pi * (d * (1.0 / CUTOFF))) + 0.5
        sw_ref[...] = jnp.where(edge, sw, 0.0)
        m_ref[...] = edge

    return pl.pallas_call(
        merge_body,
        grid=(grid,),
        in_specs=[pl.BlockSpec((TC_BLK,), lambda i: (i,))] * 4,
        out_specs=[
            pl.BlockSpec((3, TC_BLK), lambda i: (0, i)),
            pl.BlockSpec((TC_BLK,), lambda i: (i,)),
            pl.BlockSpec((TC_BLK,), lambda i: (i,)),
        ],
        out_shape=[
            jax.ShapeDtypeStruct((3, e_f), jnp.float32),
            jax.ShapeDtypeStruct((e_f,), jnp.float32),
            jax.ShapeDtypeStruct((e_f,), jnp.bool_),
        ],
    )


def kernel(vec, distances, filter_indices):
    e_in = vec.shape[0]
    e_f = filter_indices.shape[0]
    nc, ns = _num_workers()
    nw = nc * ns
    assert e_f % KB == 0, e_f
    nblk = e_f // KB
    tmax = pl.cdiv(nblk, nw)
    ngrp = KB // LANES

    def body(p0_hbm, p1_hbm, p2_hbm, dist_hbm, fidx_hbm, vf0_hbm, vf1_hbm,
             vf2_hbm, distf_hbm, idx_b, cidx_b, dist_b, vc0_b, vc1_b,
             vc2_b, sem):
        wid = lax.axis_index("s") * nc + lax.axis_index("c")
        e_in_v = _splat(e_in, jnp.int32)
        neg1 = _splat(-1, jnp.int32)
        fill_v = _splat(CUTOFF)

        def block_body(t, carry):
            b = wid + nw * t

            @pl.when(b < nblk)
            def _():
                base = b * KB
                pltpu.sync_copy(fidx_hbm.at[pl.ds(base, KB)], idx_b)

                def clamp_body(j, c):
                    s16 = pl.ds(j * LANES, LANES)
                    iv = idx_b[s16]
                    cidx_b[s16] = jnp.where(iv < e_in_v, iv, neg1)
                    return c

                lax.fori_loop(0, ngrp, clamp_body, 0)

                idx = plsc.Indices(cidx_b, ignored_value=-1)
                cp = [
                    pltpu.async_copy(dist_hbm.at[idx], dist_b, sem),
                    pltpu.async_copy(p0_hbm.at[idx], vc0_b, sem),
                    pltpu.async_copy(p1_hbm.at[idx], vc1_b, sem),
                    pltpu.async_copy(p2_hbm.at[idx], vc2_b, sem),
                ]
                for c in cp:
                    c.wait()

                def post_body(j, c):
                    s16 = pl.ds(j * LANES, LANES)
                    oob = idx_b[s16] >= e_in_v
                    dist_b[s16] = jnp.where(oob, fill_v, dist_b[s16])
                    vc0_b[s16] = jnp.where(oob, fill_v, vc0_b[s16])
                    vc1_b[s16] = jnp.where(oob, fill_v, vc1_b[s16])
                    vc2_b[s16] = jnp.where(oob, fill_v, vc2_b[s16])
                    return c

                lax.fori_loop(0, ngrp, post_body, 0)

                pltpu.sync_copy(dist_b, distf_hbm.at[pl.ds(base, KB)])
                pltpu.sync_copy(vc0_b, vf0_hbm.at[pl.ds(base, KB)])
                pltpu.sync_copy(vc1_b, vf1_hbm.at[pl.ds(base, KB)])
                pltpu.sync_copy(vc2_b, vf2_hbm.at[pl.ds(base, KB)])

            return carry

        lax.fori_loop(0, tmax, block_body, 0)

    mesh = plsc.VectorSubcoreMesh(core_axis_name="c", subcore_axis_name="s")
    run = pl.kernel(
        body,
        out_type=(
            jax.ShapeDtypeStruct((e_f,), jnp.float32),
            jax.ShapeDtypeStruct((e_f,), jnp.float32),
            jax.ShapeDtypeStruct((e_f,), jnp.float32),
            jax.ShapeDtypeStruct((e_f,), jnp.float32),
        ),
        mesh=mesh,
        compiler_params=pltpu.CompilerParams(needs_layout_passes=False),
        scratch_types=[
            pltpu.VMEM((KB,), jnp.int32),
            pltpu.VMEM((KB,), jnp.int32),
            pltpu.VMEM((KB,), jnp.float32),
            pltpu.VMEM((KB,), jnp.float32),
            pltpu.VMEM((KB,), jnp.float32),
            pltpu.VMEM((KB,), jnp.float32),
            pltpu.SemaphoreType.DMA,
        ],
    )
    p0, p1, p2 = _make_splitter(e_in)(vec.T)
    vf0, vf1, vf2, dist_f = run(p0, p1, p2, distances, filter_indices)
    vecf_t, switch, mask = _make_merger(e_f)(vf0, vf1, vf2, dist_f)
    return vecf_t.T, dist_f, switch, mask


# speedup vs baseline: 18.1247x; 1.1606x over previous
import jax
import jax.numpy as jnp
from jax import lax
from jax.experimental import pallas as pl
from jax.experimental.pallas import tpu as pltpu
from jax.experimental.pallas import tpu_sc as plsc

CUTOFF = 5.0
KB = 6400
LANES = 16
TC_BLK = 128000


def _splat(x, dtype=jnp.float32):
    return lax.broadcast_in_dim(jnp.asarray(x, dtype), (LANES,), ())


def _num_workers():
    try:
        info = plsc.get_sparse_core_info()
        return info.num_cores, info.num_subcores
    except Exception:
        return 2, 16


def _make_splitter(e_in):
    grid = pl.cdiv(e_in, TC_BLK)

    def split_body(vt_ref, p0_ref, p1_ref, p2_ref):
        x = vt_ref[...]
        p0_ref[...] = x[0]
        p1_ref[...] = x[1]
        p2_ref[...] = x[2]

    return pl.pallas_call(
        split_body,
        grid=(grid,),
        in_specs=[pl.BlockSpec((3, TC_BLK), lambda i: (0, i))],
        out_specs=[
            pl.BlockSpec((TC_BLK,), lambda i: (i,)),
            pl.BlockSpec((TC_BLK,), lambda i: (i,)),
            pl.BlockSpec((TC_BLK,), lambda i: (i,)),
        ],
        out_shape=[jax.ShapeDtypeStruct((e_in,), jnp.float32)] * 3,
    )


def _make_merger(e_f):
    grid = pl.cdiv(e_f, TC_BLK)

    def merge_body(v0_ref, v1_ref, v2_ref, d_ref, vt_ref, sw_ref, m_ref):
        vt_ref[...] = jnp.stack(
            [v0_ref[...], v1_ref[...], v2_ref[...]], axis=0)
        d = d_ref[...]
        edge = d < CUTOFF
        sw = 0.5 * jnp.cos(jnp.pi * (d * (1.0 / CUTOFF))) + 0.5
        sw_ref[...] = jnp.where(edge, sw, 0.0)
        m_ref[...] = edge

    return pl.pallas_call(
        merge_body,
        grid=(grid,),
        in_specs=[pl.BlockSpec((TC_BLK,), lambda i: (i,))] * 4,
        out_specs=[
            pl.BlockSpec((3, TC_BLK), lambda i: (0, i)),
            pl.BlockSpec((TC_BLK,), lambda i: (i,)),
            pl.BlockSpec((TC_BLK,), lambda i: (i,)),
        ],
        out_shape=[
            jax.ShapeDtypeStruct((3, e_f), jnp.float32),
            jax.ShapeDtypeStruct((e_f,), jnp.float32),
            jax.ShapeDtypeStruct((e_f,), jnp.bool_),
        ],
    )


def kernel(vec, distances, filter_indices):
    e_in = vec.shape[0]
    e_f = filter_indices.shape[0]
    nc, ns = _num_workers()
    nw = nc * ns
    assert e_f % KB == 0, e_f
    nblk = e_f // KB
    tmax = pl.cdiv(nblk, nw)
    ngrp = KB // LANES

    assert tmax % 2 == 0, tmax

    def body(p0_hbm, p1_hbm, p2_hbm, dist_hbm, fidx_hbm, vf0_hbm, vf1_hbm,
             vf2_hbm, distf_hbm, idx_b0, idx_b1, cidx_b0, cidx_b1,
             dist_b0, dist_b1, vc0_b0, vc0_b1, vc1_b0, vc1_b1, vc2_b0,
             vc2_b1, sem_g0, sem_g1, sem_w0, sem_w1):
        wid = lax.axis_index("s") * nc + lax.axis_index("c")
        e_in_v = _splat(e_in, jnp.int32)
        neg1 = _splat(-1, jnp.int32)
        fill_v = _splat(CUTOFF)
        idx_b = (idx_b0, idx_b1)
        cidx_b = (cidx_b0, cidx_b1)
        sem_g = (sem_g0, sem_g1)
        sem_w = (sem_w0, sem_w1)
        bufs = (
            ((dist_b0, distf_hbm), (vc0_b0, vf0_hbm), (vc1_b0, vf1_hbm),
             (vc2_b0, vf2_hbm)),
            ((dist_b1, distf_hbm), (vc0_b1, vf0_hbm), (vc1_b1, vf1_hbm),
             (vc2_b1, vf2_hbm)),
        )
        gsrc = (
            ((dist_b0, dist_hbm), (vc0_b0, p0_hbm), (vc1_b0, p1_hbm),
             (vc2_b0, p2_hbm)),
            ((dist_b1, dist_hbm), (vc0_b1, p0_hbm), (vc1_b1, p1_hbm),
             (vc2_b1, p2_hbm)),
        )

        def stage_in(t, si, drain):
            b = wid + nw * t

            @pl.when(b < nblk)
            def _():
                base = b * KB
                pltpu.sync_copy(fidx_hbm.at[pl.ds(base, KB)], idx_b[si])

                def clamp_body(j, c):
                    s16 = pl.ds(j * LANES, LANES)
                    iv = idx_b[si][s16]
                    cidx_b[si][s16] = jnp.where(iv < e_in_v, iv, neg1)
                    return c

                lax.fori_loop(0, ngrp, clamp_body, 0)
                if drain:

                    @pl.when(t >= 2)
                    def _():
                        for buf, hb in bufs[si]:
                            pltpu.make_async_copy(
                                buf, hb.at[pl.ds(base, KB)],
                                sem_w[si]).wait()

                idx = plsc.Indices(cidx_b[si], ignored_value=-1)
                for buf, hb in gsrc[si]:
                    pltpu.async_copy(hb.at[idx], buf, sem_g[si])

        def stage_out(t, so):
            b = wid + nw * t

            @pl.when(b < nblk)
            def _():
                base = b * KB
                idx = plsc.Indices(cidx_b[so], ignored_value=-1)
                for buf, hb in gsrc[so]:
                    pltpu.make_async_copy(hb.at[idx], buf,
                                          sem_g[so]).wait()

                def post_body(j, c):
                    s16 = pl.ds(j * LANES, LANES)
                    oob = idx_b[so][s16] >= e_in_v
                    for buf, _ in bufs[so]:
                        buf[s16] = jnp.where(oob, fill_v, buf[s16])
                    return c

                lax.fori_loop(0, ngrp, post_body, 0)

                @pl.when(b + 2 * nw < nblk)
                def _():
                    for buf, hb in bufs[so]:
                        pltpu.async_copy(buf, hb.at[pl.ds(base, KB)],
                                         sem_w[so])

                @pl.when(b + 2 * nw >= nblk)
                def _():
                    for buf, hb in bufs[so]:
                        pltpu.sync_copy(buf, hb.at[pl.ds(base, KB)])

        stage_in(jnp.int32(0), 0, drain=False)

        def g_body(g, carry):
            for tt in range(2):
                t = 2 * g + tt
                stage_in(t + 1, 1 - tt, drain=True)
                stage_out(t, tt)
            return carry

        lax.fori_loop(0, tmax // 2, g_body, 0)

    mesh = plsc.VectorSubcoreMesh(core_axis_name="c", subcore_axis_name="s")
    run = pl.kernel(
        body,
        out_type=(
            jax.ShapeDtypeStruct((e_f,), jnp.float32),
            jax.ShapeDtypeStruct((e_f,), jnp.float32),
            jax.ShapeDtypeStruct((e_f,), jnp.float32),
            jax.ShapeDtypeStruct((e_f,), jnp.float32),
        ),
        mesh=mesh,
        compiler_params=pltpu.CompilerParams(needs_layout_passes=False),
        scratch_types=(
            [pltpu.VMEM((KB,), jnp.int32)] * 4
            + [pltpu.VMEM((KB,), jnp.float32)] * 8
            + [pltpu.SemaphoreType.DMA] * 4
        ),
    )
    p0, p1, p2 = _make_splitter(e_in)(vec.T)
    vf0, vf1, vf2, dist_f = run(p0, p1, p2, distances, filter_indices)
    vecf_t, switch, mask = _make_merger(e_f)(vf0, vf1, vf2, dist_f)
    return vecf_t.T, dist_f, switch, mask


# speedup vs baseline: 18.1848x; 1.0033x over previous
import jax
import jax.numpy as jnp
from jax import lax
from jax.experimental import pallas as pl
from jax.experimental.pallas import tpu as pltpu
from jax.experimental.pallas import tpu_sc as plsc

CUTOFF = 5.0
KB = 10000
LANES = 16
TC_BLK = 128000


def _splat(x, dtype=jnp.float32):
    return lax.broadcast_in_dim(jnp.asarray(x, dtype), (LANES,), ())


def _num_workers():
    try:
        info = plsc.get_sparse_core_info()
        return info.num_cores, info.num_subcores
    except Exception:
        return 2, 16


def _make_splitter(e_in):
    grid = pl.cdiv(e_in, TC_BLK)

    def split_body(vt_ref, p0_ref, p1_ref, p2_ref):
        x = vt_ref[...]
        p0_ref[...] = x[0]
        p1_ref[...] = x[1]
        p2_ref[...] = x[2]

    return pl.pallas_call(
        split_body,
        grid=(grid,),
        in_specs=[pl.BlockSpec((3, TC_BLK), lambda i: (0, i))],
        out_specs=[
            pl.BlockSpec((TC_BLK,), lambda i: (i,)),
            pl.BlockSpec((TC_BLK,), lambda i: (i,)),
            pl.BlockSpec((TC_BLK,), lambda i: (i,)),
        ],
        out_shape=[jax.ShapeDtypeStruct((e_in,), jnp.float32)] * 3,
    )


def _make_merger(e_f):
    grid = pl.cdiv(e_f, TC_BLK)

    def merge_body(v0_ref, v1_ref, v2_ref, d_ref, vt_ref, sw_ref, m_ref):
        vt_ref[...] = jnp.stack(
            [v0_ref[...], v1_ref[...], v2_ref[...]], axis=0)
        d = d_ref[...]
        edge = d < CUTOFF
        sw = 0.5 * jnp.cos(jnp.pi * (d * (1.0 / CUTOFF))) + 0.5
        sw_ref[...] = jnp.where(edge, sw, 0.0)
        m_ref[...] = edge

    return pl.pallas_call(
        merge_body,
        grid=(grid,),
        in_specs=[pl.BlockSpec((TC_BLK,), lambda i: (i,))] * 4,
        out_specs=[
            pl.BlockSpec((3, TC_BLK), lambda i: (0, i)),
            pl.BlockSpec((TC_BLK,), lambda i: (i,)),
            pl.BlockSpec((TC_BLK,), lambda i: (i,)),
        ],
        out_shape=[
            jax.ShapeDtypeStruct((3, e_f), jnp.float32),
            jax.ShapeDtypeStruct((e_f,), jnp.float32),
            jax.ShapeDtypeStruct((e_f,), jnp.bool_),
        ],
    )


def kernel(vec, distances, filter_indices):
    e_in = vec.shape[0]
    e_f = filter_indices.shape[0]
    nc, ns = _num_workers()
    nw = nc * ns
    assert e_f % KB == 0, e_f
    nblk = e_f // KB
    tmax = pl.cdiv(nblk, nw)
    ngrp = KB // LANES

    assert tmax % 2 == 0, tmax

    def body(p0_hbm, p1_hbm, p2_hbm, dist_hbm, fidx_hbm, vf0_hbm, vf1_hbm,
             vf2_hbm, distf_hbm, idx_b0, idx_b1, cidx_b0, cidx_b1,
             dist_b0, dist_b1, vc0_b0, vc0_b1, vc1_b0, vc1_b1, vc2_b0,
             vc2_b1, sem_g0, sem_g1, sem_w0, sem_w1):
        wid = lax.axis_index("s") * nc + lax.axis_index("c")
        e_in_v = _splat(e_in, jnp.int32)
        neg1 = _splat(-1, jnp.int32)
        fill_v = _splat(CUTOFF)
        idx_b = (idx_b0, idx_b1)
        cidx_b = (cidx_b0, cidx_b1)
        sem_g = (sem_g0, sem_g1)
        sem_w = (sem_w0, sem_w1)
        bufs = (
            ((dist_b0, distf_hbm), (vc0_b0, vf0_hbm), (vc1_b0, vf1_hbm),
             (vc2_b0, vf2_hbm)),
            ((dist_b1, distf_hbm), (vc0_b1, vf0_hbm), (vc1_b1, vf1_hbm),
             (vc2_b1, vf2_hbm)),
        )
        gsrc = (
            ((dist_b0, dist_hbm), (vc0_b0, p0_hbm), (vc1_b0, p1_hbm),
             (vc2_b0, p2_hbm)),
            ((dist_b1, dist_hbm), (vc0_b1, p0_hbm), (vc1_b1, p1_hbm),
             (vc2_b1, p2_hbm)),
        )

        def stage_in(t, si, drain):
            b = wid + nw * t

            @pl.when(b < nblk)
            def _():
                base = b * KB
                pltpu.sync_copy(fidx_hbm.at[pl.ds(base, KB)], idx_b[si])

                def clamp_body(j, c):
                    s16 = pl.ds(j * LANES, LANES)
                    iv = idx_b[si][s16]
                    cidx_b[si][s16] = jnp.where(iv < e_in_v, iv, neg1)
                    return c

                lax.fori_loop(0, ngrp, clamp_body, 0)
                if drain:

                    @pl.when(t >= 2)
                    def _():
                        for buf, hb in bufs[si]:
                            pltpu.make_async_copy(
                                buf, hb.at[pl.ds(base, KB)],
                                sem_w[si]).wait()

                idx = plsc.Indices(cidx_b[si], ignored_value=-1)
                for buf, hb in gsrc[si]:
                    pltpu.async_copy(hb.at[idx], buf, sem_g[si])

        def stage_out(t, so):
            b = wid + nw * t

            @pl.when(b < nblk)
            def _():
                base = b * KB
                idx = plsc.Indices(cidx_b[so], ignored_value=-1)
                for buf, hb in gsrc[so]:
                    pltpu.make_async_copy(hb.at[idx], buf,
                                          sem_g[so]).wait()

                def post_body(j, c):
                    s16 = pl.ds(j * LANES, LANES)
                    oob = idx_b[so][s16] >= e_in_v
                    for buf, _ in bufs[so]:
                        buf[s16] = jnp.where(oob, fill_v, buf[s16])
                    return c

                lax.fori_loop(0, ngrp, post_body, 0)

                @pl.when(b + 2 * nw < nblk)
                def _():
                    for buf, hb in bufs[so]:
                        pltpu.async_copy(buf, hb.at[pl.ds(base, KB)],
                                         sem_w[so])

                @pl.when(b + 2 * nw >= nblk)
                def _():
                    for buf, hb in bufs[so]:
                        pltpu.sync_copy(buf, hb.at[pl.ds(base, KB)])

        stage_in(jnp.int32(0), 0, drain=False)

        def g_body(g, carry):
            for tt in range(2):
                t = 2 * g + tt
                stage_in(t + 1, 1 - tt, drain=True)
                stage_out(t, tt)
            return carry

        lax.fori_loop(0, tmax // 2, g_body, 0)

    mesh = plsc.VectorSubcoreMesh(core_axis_name="c", subcore_axis_name="s")
    run = pl.kernel(
        body,
        out_type=(
            jax.ShapeDtypeStruct((e_f,), jnp.float32),
            jax.ShapeDtypeStruct((e_f,), jnp.float32),
            jax.ShapeDtypeStruct((e_f,), jnp.float32),
            jax.ShapeDtypeStruct((e_f,), jnp.float32),
        ),
        mesh=mesh,
        compiler_params=pltpu.CompilerParams(needs_layout_passes=False),
        scratch_types=(
            [pltpu.VMEM((KB,), jnp.int32)] * 4
            + [pltpu.VMEM((KB,), jnp.float32)] * 8
            + [pltpu.SemaphoreType.DMA] * 4
        ),
    )
    p0, p1, p2 = _make_splitter(e_in)(vec.T)
    vf0, vf1, vf2, dist_f = run(p0, p1, p2, distances, filter_indices)
    vecf_t, switch, mask = _make_merger(e_f)(vf0, vf1, vf2, dist_f)
    return vecf_t.T, dist_f, switch, mask
